# baseline (speedup 1.0000x reference)
import functools

import jax
import jax.numpy as jnp
from jax.experimental import pallas as pl
from jax.experimental.pallas import tpu as pltpu

_HIDDEN = 128
_D_IN = 28 * 28
_PHASOR_H = 32
_DELTA0 = 7.0
_BLK = 256
_NBUF = 4


def _fused_kernel(x_hbm_ref, W1_ref, b1_ref, Wp_ref, bp_ref, Wo_ref, bo_ref,
                  out_ref, xbuf_ref, copy_sems):
    i = pl.program_id(0)
    nsteps = pl.num_programs(0)

    def _copy(j):
        slot = jax.lax.rem(j, _NBUF)
        return pltpu.make_async_copy(
            x_hbm_ref.at[pl.ds(j * _BLK, _BLK)],
            xbuf_ref.at[slot],
            copy_sems.at[slot])

    @pl.when(i == 0)
    def _():
        for k in range(_NBUF - 1):
            @pl.when(k < nsteps)
            def _():
                _copy(k).start()

    @pl.when(i + _NBUF - 1 < nsteps)
    def _():
        _copy(i + _NBUF - 1).start()

    _copy(i).wait()
    x3 = xbuf_ref[jax.lax.rem(i, _NBUF)]
    out_ref[...] = x3[:, 0, :10] + W1_ref[0, 0]
    return
    projected = jnp.dot(x, W1_ref[...],
                        preferred_element_type=jnp.float32) + b1_ref[...]
    x_mean = jnp.mean(projected, axis=-1, keepdims=True)
    h = jax.lax.broadcasted_iota(jnp.int32, (1, _PHASOR_H), 1).astype(
        jnp.float32) + 1.0
    phase = x_mean * (_DELTA0 * h)
    feats = jnp.concatenate([jnp.cos(phase), jnp.sin(phase)], axis=-1)
    temporal = jnp.dot(feats, Wp_ref[...],
                       preferred_element_type=jnp.float32) + bp_ref[...]
    enhanced = projected + temporal
    out_ref[...] = jnp.dot(enhanced, Wo_ref[...],
                           preferred_element_type=jnp.float32) + bo_ref[...]


@functools.partial(jax.jit, static_argnames=())
def kernel(x, W1, b1, Wp, bp, Wo, bo):
    B = x.shape[0]
    n_out = Wo.shape[1]
    grid = (B // _BLK,)
    return pl.pallas_call(
        _fused_kernel,
        grid=grid,
        in_specs=[
            pl.BlockSpec(memory_space=pl.MemorySpace.ANY),
            pl.BlockSpec((_D_IN, _HIDDEN), lambda i: (0, 0)),
            pl.BlockSpec((1, _HIDDEN), lambda i: (0, 0)),
            pl.BlockSpec((2 * _PHASOR_H, _HIDDEN), lambda i: (0, 0)),
            pl.BlockSpec((1, _HIDDEN), lambda i: (0, 0)),
            pl.BlockSpec((_HIDDEN, n_out), lambda i: (0, 0)),
            pl.BlockSpec((1, n_out), lambda i: (0, 0)),
        ],
        out_specs=pl.BlockSpec((_BLK, n_out), lambda i: (i, 0)),
        out_shape=jax.ShapeDtypeStruct((B, n_out), jnp.float32),
        scratch_shapes=[
            pltpu.VMEM((_NBUF, _BLK, 28, 28), jnp.float32),
            pltpu.SemaphoreType.DMA((_NBUF,)),
        ],
        compiler_params=pltpu.CompilerParams(
            dimension_semantics=("arbitrary",),
        ),
    )(x, W1, b1.reshape(1, -1), Wp, bp.reshape(1, -1),
      Wo, bo.reshape(1, -1))


# speedup vs baseline: 1.9063x; 1.9063x over previous
import functools

import jax
import jax.numpy as jnp
from jax.experimental import pallas as pl
from jax.experimental.pallas import tpu as pltpu

_HIDDEN = 128
_D_IN = 28 * 28
_PHASOR_H = 32
_DELTA0 = 7.0
_BLK = 1024
_NBUF = 4


def _fused_kernel(x_hbm_ref, W1_ref, b1_ref, Wp_ref, bp_ref, Wo_ref, bo_ref,
                  out_ref, xbuf_ref, copy_sems):
    i = pl.program_id(0)
    nsteps = pl.num_programs(0)

    def _copy(j):
        slot = jax.lax.rem(j, _NBUF)
        return pltpu.make_async_copy(
            x_hbm_ref.at[pl.ds(j * _BLK, _BLK), :],
            xbuf_ref.at[slot],
            copy_sems.at[slot])

    @pl.when(i == 0)
    def _():
        for k in range(_NBUF - 1):
            @pl.when(k < nsteps)
            def _():
                _copy(k).start()

    @pl.when(i + _NBUF - 1 < nsteps)
    def _():
        _copy(i + _NBUF - 1).start()

    _copy(i).wait()
    x = xbuf_ref[jax.lax.rem(i, _NBUF)]
    projected = jnp.dot(x, W1_ref[...],
                        preferred_element_type=jnp.float32) + b1_ref[...]
    x_mean = jnp.mean(projected, axis=-1, keepdims=True)
    h = jax.lax.broadcasted_iota(jnp.int32, (1, _PHASOR_H), 1).astype(
        jnp.float32) + 1.0
    phase = x_mean * (_DELTA0 * h)
    feats = jnp.concatenate([jnp.cos(phase), jnp.sin(phase)], axis=-1)
    temporal = jnp.dot(feats, Wp_ref[...],
                       preferred_element_type=jnp.float32) + bp_ref[...]
    enhanced = projected + temporal
    out_ref[...] = jnp.dot(enhanced, Wo_ref[...],
                           preferred_element_type=jnp.float32) + bo_ref[...]


@functools.partial(jax.jit, static_argnames=())
def kernel(x, W1, b1, Wp, bp, Wo, bo):
    B = x.shape[0]
    x_flat = x.reshape(B, _D_IN)
    n_out = Wo.shape[1]
    grid = (B // _BLK,)
    return pl.pallas_call(
        _fused_kernel,
        grid=grid,
        in_specs=[
            pl.BlockSpec(memory_space=pl.MemorySpace.ANY),
            pl.BlockSpec((_D_IN, _HIDDEN), lambda i: (0, 0)),
            pl.BlockSpec((1, _HIDDEN), lambda i: (0, 0)),
            pl.BlockSpec((2 * _PHASOR_H, _HIDDEN), lambda i: (0, 0)),
            pl.BlockSpec((1, _HIDDEN), lambda i: (0, 0)),
            pl.BlockSpec((_HIDDEN, n_out), lambda i: (0, 0)),
            pl.BlockSpec((1, n_out), lambda i: (0, 0)),
        ],
        out_specs=pl.BlockSpec((_BLK, n_out), lambda i: (i, 0)),
        out_shape=jax.ShapeDtypeStruct((B, n_out), jnp.float32),
        scratch_shapes=[
            pltpu.VMEM((_NBUF, _BLK, _D_IN), jnp.float32),
            pltpu.SemaphoreType.DMA((_NBUF,)),
        ],
        compiler_params=pltpu.CompilerParams(
            dimension_semantics=("parallel",),
        ),
    )(x_flat, W1, b1.reshape(1, -1), Wp, bp.reshape(1, -1),
      Wo, bo.reshape(1, -1))


# speedup vs baseline: 2.2888x; 1.2006x over previous
import functools

import jax
import jax.numpy as jnp
from jax.experimental import pallas as pl
from jax.experimental.pallas import tpu as pltpu

_HIDDEN = 128
_D_IN = 28 * 28
_PHASOR_H = 32
_DELTA0 = 7.0
_BLK = 1024
_NBUF = 4


def _fused_kernel(x_hbm_ref, W1_ref, b1_ref, Wp_ref, bp_ref, Wo_ref, bo_ref,
                  out_ref, xbuf_ref, copy_sems):
    i = pl.program_id(0)
    nsteps = pl.num_programs(0)

    def _copy(j):
        slot = jax.lax.rem(j, _NBUF)
        return pltpu.make_async_copy(
            x_hbm_ref.at[pl.ds(j * _BLK, _BLK), :],
            xbuf_ref.at[slot],
            copy_sems.at[slot])

    @pl.when(i == 0)
    def _():
        for k in range(_NBUF - 1):
            @pl.when(k < nsteps)
            def _():
                _copy(k).start()

    @pl.when(i + _NBUF - 1 < nsteps)
    def _():
        _copy(i + _NBUF - 1).start()

    _copy(i).wait()
    x = xbuf_ref[jax.lax.rem(i, _NBUF)]
    projected = jnp.dot(x, W1_ref[...],
                        preferred_element_type=jnp.float32) + b1_ref[...]
    x_mean = jnp.mean(projected, axis=-1, keepdims=True)

    half = _BLK // 2
    lane = jax.lax.broadcasted_iota(jnp.int32, (1, 2 * _HIDDEN // 2), 1)
    f = lane % 64
    mvec = (_DELTA0 * ((f % _PHASOR_H) + 1)).astype(jnp.float32)
    svec = jnp.where(f >= _PHASOR_H, 1.5707963705062866, 0.0).astype(
        jnp.float32)
    xm = jnp.where(lane < 64, x_mean[:half], x_mean[half:])
    z = xm * mvec - svec
    k = jnp.rint(z * 0.6366197723675814)
    r = z - k * 1.5707963705062866
    r = r - k * (-4.371139000186241e-08)
    q = k.astype(jnp.int32) & 3
    r2 = r * r
    cr = 1.0 + r2 * (-0.5 + r2 * (0.041666668 + r2 * (
        -0.0013888889 + r2 * 2.48016e-05)))
    sr = r * (1.0 + r2 * (-0.16666667 + r2 * (0.008333331 + r2 * (
        -0.000198409 + r2 * 2.7526e-06))))
    val = jnp.where((q & 1) == 1, sr, cr)
    feats2 = jnp.where((q == 1) | (q == 2), -val, val)

    temporal2 = jnp.dot(feats2, Wp_ref[...],
                        preferred_element_type=jnp.float32)
    temporal = jnp.concatenate(
        [temporal2[:, :_HIDDEN], temporal2[:, _HIDDEN:]], axis=0)
    enhanced = projected + temporal + bp_ref[...]
    out_ref[...] = jnp.dot(enhanced, Wo_ref[...],
                           preferred_element_type=jnp.float32) + bo_ref[...]


@functools.partial(jax.jit, static_argnames=())
def kernel(x, W1, b1, Wp, bp, Wo, bo):
    B = x.shape[0]
    x_flat = x.reshape(B, _D_IN)
    n_out = Wo.shape[1]
    grid = (B // _BLK,)
    call = pl.pallas_call(
        _fused_kernel,
        grid=grid,
        in_specs=[
            pl.BlockSpec(memory_space=pl.MemorySpace.ANY),
            pl.BlockSpec((_D_IN, _HIDDEN), lambda i: (0, 0)),
            pl.BlockSpec((1, _HIDDEN), lambda i: (0, 0)),
            pl.BlockSpec((2 * _PHASOR_H * 2, 2 * _HIDDEN), lambda i: (0, 0)),
            pl.BlockSpec((1, _HIDDEN), lambda i: (0, 0)),
            pl.BlockSpec((_HIDDEN, n_out), lambda i: (0, 0)),
            pl.BlockSpec((1, n_out), lambda i: (0, 0)),
        ],
        out_specs=pl.BlockSpec((_BLK, n_out), lambda i: (i, 0)),
        out_shape=jax.ShapeDtypeStruct((B, n_out), jnp.float32),
        scratch_shapes=[
            pltpu.VMEM((_NBUF, _BLK, _D_IN), jnp.float32),
            pltpu.SemaphoreType.DMA((_NBUF,)),
        ],
        compiler_params=pltpu.CompilerParams(
            dimension_semantics=("arbitrary",),
        ),
    )
    Wp2 = jnp.zeros((2 * 2 * _PHASOR_H, 2 * _HIDDEN), dtype=Wp.dtype)
    Wp2 = Wp2.at[:2 * _PHASOR_H, :_HIDDEN].set(Wp)
    Wp2 = Wp2.at[2 * _PHASOR_H:, _HIDDEN:].set(Wp)
    return call(x_flat, W1, b1.reshape(1, -1), Wp2, bp.reshape(1, -1),
                Wo, bo.reshape(1, -1))


# speedup vs baseline: 2.3446x; 1.0244x over previous
import functools

import jax
import jax.numpy as jnp
from jax.experimental import pallas as pl
from jax.experimental.pallas import tpu as pltpu

_HIDDEN = 128
_D_IN = 28 * 28
_PHASOR_H = 32
_DELTA0 = 7.0
_BLK = 2048
_NBUF = 4


def _fused_kernel(x_hbm_ref, W1_ref, b1_ref, Wp_ref, bp_ref, Wo_ref, bo_ref,
                  out_ref, xbuf_ref, copy_sems):
    i = pl.program_id(0)
    nsteps = pl.num_programs(0)

    def _copy(j):
        slot = jax.lax.rem(j, _NBUF)
        return pltpu.make_async_copy(
            x_hbm_ref.at[pl.ds(j * _BLK, _BLK), :],
            xbuf_ref.at[slot],
            copy_sems.at[slot])

    @pl.when(i == 0)
    def _():
        for k in range(_NBUF - 1):
            @pl.when(k < nsteps)
            def _():
                _copy(k).start()

    @pl.when(i + _NBUF - 1 < nsteps)
    def _():
        _copy(i + _NBUF - 1).start()

    _copy(i).wait()
    x = xbuf_ref[jax.lax.rem(i, _NBUF)]
    projected = jnp.dot(x, W1_ref[...],
                        preferred_element_type=jnp.float32) + b1_ref[...]
    x_mean = jnp.mean(projected, axis=-1, keepdims=True)

    half = _BLK // 2
    lane = jax.lax.broadcasted_iota(jnp.int32, (1, 2 * _HIDDEN // 2), 1)
    f = lane % 64
    mvec = (_DELTA0 * ((f % _PHASOR_H) + 1)).astype(jnp.float32)
    svec = jnp.where(f >= _PHASOR_H, 1.5707963705062866, 0.0).astype(
        jnp.float32)
    xm = jnp.where(lane < 64, x_mean[:half], x_mean[half:])
    z = xm * mvec - svec
    k = jnp.rint(z * 0.6366197723675814)
    r = z - k * 1.5707963705062866
    r = r - k * (-4.371139000186241e-08)
    q = k.astype(jnp.int32) & 3
    r2 = r * r
    cr = 1.0 + r2 * (-0.5 + r2 * (0.041666668 + r2 * (
        -0.0013888889 + r2 * 2.48016e-05)))
    sr = r * (1.0 + r2 * (-0.16666667 + r2 * (0.008333331 + r2 * (
        -0.000198409 + r2 * 2.7526e-06))))
    val = jnp.where((q & 1) == 1, sr, cr)
    feats2 = jnp.where((q == 1) | (q == 2), -val, val)

    temporal2 = jnp.dot(feats2, Wp_ref[...],
                        preferred_element_type=jnp.float32)
    temporal = jnp.concatenate(
        [temporal2[:, :_HIDDEN], temporal2[:, _HIDDEN:]], axis=0)
    enhanced = projected + temporal + bp_ref[...]
    out_ref[...] = jnp.dot(enhanced, Wo_ref[...],
                           preferred_element_type=jnp.float32) + bo_ref[...]


@functools.partial(jax.jit, static_argnames=())
def kernel(x, W1, b1, Wp, bp, Wo, bo):
    B = x.shape[0]
    x_flat = x.reshape(B, _D_IN)
    n_out = Wo.shape[1]
    grid = (B // _BLK,)
    call = pl.pallas_call(
        _fused_kernel,
        grid=grid,
        in_specs=[
            pl.BlockSpec(memory_space=pl.MemorySpace.ANY),
            pl.BlockSpec((_D_IN, _HIDDEN), lambda i: (0, 0)),
            pl.BlockSpec((1, _HIDDEN), lambda i: (0, 0)),
            pl.BlockSpec((2 * _PHASOR_H * 2, 2 * _HIDDEN), lambda i: (0, 0)),
            pl.BlockSpec((1, _HIDDEN), lambda i: (0, 0)),
            pl.BlockSpec((_HIDDEN, n_out), lambda i: (0, 0)),
            pl.BlockSpec((1, n_out), lambda i: (0, 0)),
        ],
        out_specs=pl.BlockSpec((_BLK, n_out), lambda i: (i, 0)),
        out_shape=jax.ShapeDtypeStruct((B, n_out), jnp.float32),
        scratch_shapes=[
            pltpu.VMEM((_NBUF, _BLK, _D_IN), jnp.float32),
            pltpu.SemaphoreType.DMA((_NBUF,)),
        ],
        compiler_params=pltpu.CompilerParams(
            dimension_semantics=("arbitrary",),
        ),
    )
    Wp2 = jnp.zeros((2 * 2 * _PHASOR_H, 2 * _HIDDEN), dtype=Wp.dtype)
    Wp2 = Wp2.at[:2 * _PHASOR_H, :_HIDDEN].set(Wp)
    Wp2 = Wp2.at[2 * _PHASOR_H:, _HIDDEN:].set(Wp)
    return call(x_flat, W1, b1.reshape(1, -1), Wp2, bp.reshape(1, -1),
                Wo, bo.reshape(1, -1))
